# Initial kernel scaffold; baseline (speedup 1.0000x reference)
import jax
import jax.numpy as jnp
from jax.experimental import pallas as pl


def kernel(input, input_fc, input_sc, Wm, bm, Wpy, bpy, Wpx, bpx, Wq, Wk, Wv, Wo, Wro, bro, Wout, bout):
    raise NotImplementedError("write your pallas kernel here")


# speedup vs baseline: 1.6853x; 1.6853x over previous
import functools
import jax
import jax.numpy as jnp
import numpy as np
from jax import lax
from jax.experimental import pallas as pl
from jax.experimental.pallas import tpu as pltpu

N = 2048
F = 128
CH = 128
NOSC = 4
NG = CH // NOSC
H = 8
DH = CH // H
QSTEPS = 4
E = 32768
GST = 4
NCLS = 4
GAMMA = 1.0


def _encoder_body(x_ref, Wm_ref, bm_ref, WpyT_ref, bpy_ref, Wout_ref, bout_ref,
                  y_ref, logits_ref):
    x = x_ref[...]
    y = jnp.zeros((N, CH), jnp.float32) + bpy_ref[...][None, :]
    for k in range(GST):
        ys = jnp.maximum(
            jax.lax.dot_general(x, Wm_ref[k], (((1,), (0,)), ((), ())),
                                preferred_element_type=jnp.float32)
            + bm_ref[k][None, :], 0.0)
        y = y + jax.lax.dot_general(ys, WpyT_ref[pl.ds(k * F, F), :],
                                    (((1,), (0,)), ((), ())),
                                    preferred_element_type=jnp.float32)
    y_ref[...] = y
    logits_ref[...] = jax.lax.dot_general(
        y, Wout_ref[...], (((1,), (0,)), ((), ())),
        preferred_element_type=jnp.float32) + bout_ref[...][None, :]


def _encoder(x, Wm, bm, WpyT, bpy, Wout, bout):
    return pl.pallas_call(
        _encoder_body,
        out_shape=(jax.ShapeDtypeStruct((N, CH), jnp.float32),
                   jax.ShapeDtypeStruct((N, NCLS), jnp.float32)),
    )(x, Wm, bm, WpyT, bpy, Wout, bout)


def _kuramoto_body(feats_ref, yt_ref, WpxT_ref, bpx_ref, Wq_ref, Wk_ref,
                   Wv_ref, Wo_ref, Gm_ref, WroS_ref, bro_ref, xout_ref):
    feats = feats_ref[...]
    x0 = jax.lax.dot_general(feats, WpxT_ref[...], (((1,), (0,)), ((), ())),
                             preferred_element_type=jnp.float32) \
        + bpx_ref[...][None, :]
    Gm = Gm_ref[...]

    def gsum(v):
        return jax.lax.dot_general(v, Gm, (((1,), (0,)), ((), ())),
                                   preferred_element_type=jnp.float32)

    x = x0 * jax.lax.rsqrt(gsum(x0 * x0) + 1e-6)
    yt = yt_ref[...]
    scale = 1.0 / np.sqrt(DH)
    for _ in range(QSTEPS):
        Q = jax.lax.dot_general(x, Wq_ref[...], (((1,), (0,)), ((), ())),
                                preferred_element_type=jnp.float32)
        K = jax.lax.dot_general(x, Wk_ref[...], (((1,), (0,)), ((), ())),
                                preferred_element_type=jnp.float32)
        V = jax.lax.dot_general(x, Wv_ref[...], (((1,), (0,)), ((), ())),
                                preferred_element_type=jnp.float32)
        outs = []
        for h in range(H):
            Qh = Q[:, h * DH:(h + 1) * DH]
            Kh = K[:, h * DH:(h + 1) * DH]
            Vh = V[:, h * DH:(h + 1) * DH]
            S = jax.lax.dot_general(Qh, Kh, (((1,), (1,)), ((), ())),
                                    preferred_element_type=jnp.float32) * scale
            S = S - jnp.max(S, axis=-1, keepdims=True)
            Ex = jnp.exp(S)
            A = Ex / jnp.sum(Ex, axis=-1, keepdims=True)
            outs.append(jax.lax.dot_general(A, Vh, (((1,), (0,)), ((), ())),
                                            preferred_element_type=jnp.float32))
        O = jnp.concatenate(outs, axis=1)
        Jx = jax.lax.dot_general(O, Wo_ref[...], (((1,), (0,)), ((), ())),
                                 preferred_element_type=jnp.float32)
        force = Jx + yt
        dot = gsum(force * x)
        xg = x + GAMMA * (force - dot * x)
        x = xg * jax.lax.rsqrt(gsum(xg * xg) + 1e-6)
    acc = jnp.full((N, CH), 1e-6, jnp.float32)
    for o in range(NOSC):
        zo = jax.lax.dot_general(x, WroS_ref[o], (((1,), (0,)), ((), ())),
                                 preferred_element_type=jnp.float32)
        acc = acc + zo * zo
    xout_ref[...] = jnp.sqrt(acc) + bro_ref[...][None, :]


def _kuramoto(feats, yt, WpxT, bpx, Wq, Wk, Wv, Wo, Gm, WroS, bro):
    return pl.pallas_call(
        _kuramoto_body,
        out_shape=jax.ShapeDtypeStruct((N, CH), jnp.float32),
    )(feats, yt, WpxT, bpx, Wq, Wk, Wv, Wo, Gm, WroS, bro)


def kernel(input, input_fc, input_sc, Wm, bm, Wpy, bpy, Wpx, bpx, Wq, Wk, Wv,
           Wo, Wro, bro, Wout, bout):
    del input_fc
    x = input[0]
    src = input_sc[0].astype(jnp.int32)
    dst = input_sc[1].astype(jnp.int32)

    y_t, logits = _encoder(x, Wm, bm, Wpy.T, bpy, Wout, bout)

    A = jnp.zeros((N, N), jnp.float32).at[src, dst].add(1.0)
    deg = jnp.clip(jnp.sum(A, axis=0), 1.0, None)
    P = 0.5 * (jnp.eye(N, dtype=jnp.float32) + A / deg[None, :])
    d1 = P @ x
    d2 = P @ d1
    d4 = P @ (P @ d2)
    d8 = P @ (P @ (P @ (P @ d4)))
    feats = jnp.concatenate(
        [d8, jnp.abs(d1 - d2), jnp.abs(d2 - d4), jnp.abs(d4 - d8)], axis=1)

    Gm = jnp.repeat(jnp.repeat(jnp.eye(NG, dtype=jnp.float32), NOSC, axis=0),
                    NOSC, axis=1)
    WroS = jnp.stack([Wro[:, o::NOSC] for o in range(NOSC)], axis=0)
    x_out = _kuramoto(feats, y_t, Wpx.T, bpx, Wq, Wk, Wv, Wo, Gm, WroS, bro)

    logits_out = logits[None, :, :]
    x_out = x_out[None, :, :]
    saved_y = y_t.T[None, :, :]
    return logits_out, x_out, saved_y
